# baseline (speedup 1.0000x reference)
import functools

import jax
import jax.numpy as jnp
from jax import lax
from jax.experimental import pallas as pl
from jax.experimental.pallas import tpu as pltpu
from jax.experimental.pallas import tpu_sc as plsc

T, B, C, K = 512, 16, 4, 64
CK = C * K
NW = 32
TPW = T // NW
LG = 4

_F32 = jnp.float32
_I32 = jnp.int32


def _dot(a, b):
    return jnp.dot(a, b, preferred_element_type=_F32)


def _dotbf(a, b):
    return jnp.dot(a.astype(jnp.bfloat16), b.astype(jnp.bfloat16),
                   preferred_element_type=_F32)


def _num_body(em_hbm, tags_hbm, lens_hbm, trans_hbm, start_hbm, end_hbm,
              out_hbm, em_v, tg_v, trans_v, start_v, end_v, lens_v, acc_v):
    cid = lax.axis_index("c")
    sid = lax.axis_index("s")
    w = sid * 2 + cid
    t0 = w * TPW
    pltpu.sync_copy(em_hbm.at[pl.ds(w * (TPW * B * C * K), TPW * B * C * K)],
                    em_v)
    prev_off = pl.multiple_of(jnp.maximum(w * (B * C) * TPW - B * C, 0), B * C)
    pltpu.sync_copy(tags_hbm.at[pl.ds(prev_off, B * C)], tg_v.at[pl.ds(0, B * C)])
    pltpu.sync_copy(tags_hbm.at[pl.ds(w * (B * C) * TPW, B * C * TPW)],
                    tg_v.at[pl.ds(B * C, B * C * TPW)])
    pltpu.sync_copy(trans_hbm, trans_v)
    pltpu.sync_copy(start_hbm, start_v)
    pltpu.sync_copy(end_hbm, end_v)
    pltpu.sync_copy(lens_hbm, lens_v)

    lane = lax.broadcasted_iota(_I32, (16,), 0)
    consts = []
    for g in range(LG):
        p = lane + g * 16
        c = p & 3
        lens_g = lens_v[pl.ds(g * 16, 16)]
        consts.append((p * K, c * (K * K), c * K, lens_g))

    def step(t_l, accs):
        t_g = t0 + t_l
        out = []
        for g in range(LG):
            p64, c4096, c64, lens_g = consts[g]
            base = t_l * (B * C) + g * 16
            tgc = tg_v[pl.ds(base + B * C, 16)]
            tgp = tg_v[pl.ds(base, 16)]
            e = plsc.load_gather(em_v, [t_l * (B * C * K) + p64 + tgc])
            tr = plsc.load_gather(trans_v, [c4096 + tgp * K + tgc])
            en = plsc.load_gather(end_v, [c64 + tgc])
            st = plsc.load_gather(start_v, [c64 + tgc])
            maskf = jnp.where(t_g < lens_g, 1.0, 0.0).astype(_F32)
            endf = jnp.where(t_g == lens_g - 1, 1.0, 0.0).astype(_F32)
            live = jnp.where(t_g == 0, e + st, (e + tr) * maskf)
            out.append(accs[g] + live + en * endf)
        return tuple(out)

    zero = jnp.zeros((16,), _F32)
    accs = lax.fori_loop(0, TPW, step, (zero,) * LG)
    for g in range(LG):
        acc_v[pl.ds(g * 16, 16)] = accs[g]
    pltpu.sync_copy(acc_v, out_hbm.at[w])


def _sc_numerator(em_flat, tags_flat, lens, trans_flat, start_flat, end_flat):
    mesh = plsc.VectorSubcoreMesh(core_axis_name="c", subcore_axis_name="s")
    kern = functools.partial(
        pl.kernel,
        mesh=mesh,
        compiler_params=pltpu.CompilerParams(needs_layout_passes=False),
        out_type=jax.ShapeDtypeStruct((NW, B * C), _F32),
        scratch_types=[
            pltpu.VMEM((TPW * B * C * K,), _F32),
            pltpu.VMEM((B * C * (TPW + 1),), _I32),
            pltpu.VMEM((C * K * K,), _F32),
            pltpu.VMEM((C * K,), _F32),
            pltpu.VMEM((C * K,), _F32),
            pltpu.VMEM((B * C,), _I32),
            pltpu.VMEM((B * C,), _F32),
        ],
    )(_num_body)
    return kern(em_flat, tags_flat, lens, trans_flat, start_flat, end_flat)


def _den_body(em_ref, lens_ref, tlog_ref, start_ref, end_ref, out_ref):
    lane_c = lax.broadcasted_iota(_I32, (C, CK), 1) // K
    row_c = lax.broadcasted_iota(_I32, (C, CK), 0)
    BI = (lane_c == row_c).astype(_F32)
    BIT = BI.T

    E = jnp.exp(tlog_ref[...]).astype(jnp.bfloat16)
    expend = jnp.exp(end_ref[...])
    lens = lens_ref[...]

    P = jnp.exp(start_ref[...] + em_ref[0])
    norm = _dotbf(P, BIT)
    P = P * (1.0 / _dotbf(norm, BI))
    s = jnp.log(norm)
    Psnap = P
    S = s

    def light_step(t, P, s, Psnap, S):
        Pn = _dotbf(P, E) * jnp.exp(em_ref[t])
        hit = t == lens - 1
        Psnap = jnp.where(hit, Pn, Psnap)
        S = jnp.where(hit, s, S)
        return Pn, Psnap, S

    def block(i, carry):
        P, s, Psnap, S = carry
        norm = _dotbf(P, BIT)
        invb = 1.0 / _dotbf(norm, BI)
        t0 = 1 + i * 4
        for j in range(4):
            P, Psnap, S = light_step(t0 + j, P, s, Psnap, S)
        P = P * invb
        s = s + jnp.log(norm)
        return P, s, Psnap, S

    P, s, Psnap, S = lax.fori_loop(0, (T - 4) // 4, block, (P, s, Psnap, S))
    for t in range(1 + 4 * ((T - 4) // 4), T):
        P, Psnap, S = light_step(t, P, s, Psnap, S)
    out_ref[...] = S + jnp.log(_dot(Psnap * expend, BIT))


def _tc_denominator(em_t, lens2, tlogt, start_col, end_col):
    return pl.pallas_call(
        _den_body,
        out_shape=jax.ShapeDtypeStruct((B, C), _F32),
        in_specs=[pl.BlockSpec((T, B, CK), lambda: (0, 0, 0)),
                  pl.BlockSpec((B, 1), lambda: (0, 0)),
                  pl.BlockSpec((CK, CK), lambda: (0, 0)),
                  pl.BlockSpec((1, CK), lambda: (0, 0)),
                  pl.BlockSpec((1, CK), lambda: (0, 0))],
        out_specs=pl.BlockSpec((B, C), lambda: (0, 0)),
    )(em_t, lens2, tlogt, start_col, end_col)


@jax.jit
def kernel(emissions, tags, token_sizes, transitions, start_transitions,
           end_transitions):
    em = emissions.reshape(T, B, CK)
    tg32 = tags.astype(_I32)
    lens = token_sizes.astype(_I32)
    tlog = jnp.full((CK, CK), -1e30, dtype=_F32)
    for c in range(C):
        sl = slice(c * K, (c + 1) * K)
        tlog = tlog.at[sl, sl].set(transitions[c])
    start_row = start_transitions.reshape(1, CK)
    end_row = end_transitions.reshape(1, CK)

    lens_x = jnp.repeat(lens, C)
    partials = _sc_numerator(emissions.reshape(-1), tg32.reshape(-1), lens_x,
                             transitions.reshape(-1).astype(_F32),
                             start_transitions.reshape(-1),
                             end_transitions.reshape(-1))
    den = _tc_denominator(em, lens.reshape(B, 1), tlog, start_row, end_row)
    num = partials.sum(axis=0).reshape(B, C)
    return num - den


# speedup vs baseline: 1.0759x; 1.0759x over previous
import functools

import jax
import jax.numpy as jnp
from jax import lax
from jax.experimental import pallas as pl
from jax.experimental.pallas import tpu as pltpu
from jax.experimental.pallas import tpu_sc as plsc

T, B, C, K = 512, 16, 4, 64
CK = C * K
NW = 32
TPW = T // NW
LG = 4

_F32 = jnp.float32
_I32 = jnp.int32


def _dot(a, b):
    return jnp.dot(a, b, preferred_element_type=_F32)


def _dotbf(a, b):
    return jnp.dot(a.astype(jnp.bfloat16), b.astype(jnp.bfloat16),
                   preferred_element_type=_F32)


def _num_body(em_hbm, tags_hbm, lens_hbm, trans_hbm, start_hbm, end_hbm,
              out_hbm, em_v, tg_v, trans_v, start_v, end_v, lens_v, acc_v):
    cid = lax.axis_index("c")
    sid = lax.axis_index("s")
    w = sid * 2 + cid
    t0 = w * TPW
    pltpu.sync_copy(em_hbm.at[pl.ds(w * (TPW * B * C * K), TPW * B * C * K)],
                    em_v)
    prev_off = pl.multiple_of(jnp.maximum(w * (B * C) * TPW - B * C, 0), B * C)
    pltpu.sync_copy(tags_hbm.at[pl.ds(prev_off, B * C)], tg_v.at[pl.ds(0, B * C)])
    pltpu.sync_copy(tags_hbm.at[pl.ds(w * (B * C) * TPW, B * C * TPW)],
                    tg_v.at[pl.ds(B * C, B * C * TPW)])
    pltpu.sync_copy(trans_hbm, trans_v)
    pltpu.sync_copy(start_hbm, start_v)
    pltpu.sync_copy(end_hbm, end_v)
    pltpu.sync_copy(lens_hbm, lens_v)

    lane = lax.broadcasted_iota(_I32, (16,), 0)
    consts = []
    for g in range(LG):
        p = lane + g * 16
        c = p & 3
        lens_g = lens_v[pl.ds(g * 16, 16)]
        consts.append((p * K, c * (K * K), c * K, lens_g))

    def step(t_l, accs):
        t_g = t0 + t_l
        out = []
        for g in range(LG):
            p64, c4096, c64, lens_g = consts[g]
            base = t_l * (B * C) + g * 16
            tgc = tg_v[pl.ds(base + B * C, 16)]
            tgp = tg_v[pl.ds(base, 16)]
            e = plsc.load_gather(em_v, [t_l * (B * C * K) + p64 + tgc])
            tr = plsc.load_gather(trans_v, [c4096 + tgp * K + tgc])
            en = plsc.load_gather(end_v, [c64 + tgc])
            st = plsc.load_gather(start_v, [c64 + tgc])
            maskf = jnp.where(t_g < lens_g, 1.0, 0.0).astype(_F32)
            endf = jnp.where(t_g == lens_g - 1, 1.0, 0.0).astype(_F32)
            live = jnp.where(t_g == 0, e + st, (e + tr) * maskf)
            out.append(accs[g] + live + en * endf)
        return tuple(out)

    zero = jnp.zeros((16,), _F32)
    accs = lax.fori_loop(0, TPW, step, (zero,) * LG)
    for g in range(LG):
        acc_v[pl.ds(g * 16, 16)] = accs[g]
    pltpu.sync_copy(acc_v, out_hbm.at[w])


def _sc_numerator(em_flat, tags_flat, lens, trans_flat, start_flat, end_flat):
    mesh = plsc.VectorSubcoreMesh(core_axis_name="c", subcore_axis_name="s")
    kern = functools.partial(
        pl.kernel,
        mesh=mesh,
        compiler_params=pltpu.CompilerParams(needs_layout_passes=False),
        out_type=jax.ShapeDtypeStruct((NW, B * C), _F32),
        scratch_types=[
            pltpu.VMEM((TPW * B * C * K,), _F32),
            pltpu.VMEM((B * C * (TPW + 1),), _I32),
            pltpu.VMEM((C * K * K,), _F32),
            pltpu.VMEM((C * K,), _F32),
            pltpu.VMEM((C * K,), _F32),
            pltpu.VMEM((B * C,), _I32),
            pltpu.VMEM((B * C,), _F32),
        ],
    )(_num_body)
    return kern(em_flat, tags_flat, lens, trans_flat, start_flat, end_flat)


def _den_body(em_ref, lens_ref, tlog_ref, start_ref, end_ref, out_ref):
    lane_c = lax.broadcasted_iota(_I32, (C, CK), 1) // K
    row_c = lax.broadcasted_iota(_I32, (C, CK), 0)
    BI = (lane_c == row_c).astype(_F32)
    BIT = BI.T

    E = jnp.exp(tlog_ref[...]).astype(jnp.bfloat16)
    expend = jnp.exp(end_ref[...])
    lens = lens_ref[...]

    P = jnp.exp(start_ref[...] + em_ref[0])
    norm = _dotbf(P, BIT)
    P = P * (1.0 / _dotbf(norm, BI))
    s = jnp.log(norm)
    Psnap = P
    S = s

    def light_step(t, P, s, Psnap, S):
        Pn = _dotbf(P, E) * jnp.exp(em_ref[t])
        hit = t == lens - 1
        Psnap = jnp.where(hit, Pn, Psnap)
        S = jnp.where(hit, s, S)
        return Pn, Psnap, S

    def block(i, carry):
        P, s, Psnap, S = carry
        norm = _dotbf(P, BIT)
        invb = 1.0 / _dotbf(norm, BI)
        t0 = 1 + i * 8
        for j in range(8):
            P, Psnap, S = light_step(t0 + j, P, s, Psnap, S)
        P = P * invb
        s = s + jnp.log(norm)
        return P, s, Psnap, S

    P, s, Psnap, S = lax.fori_loop(0, (T - 8) // 8, block, (P, s, Psnap, S))
    for t in range(1 + 8 * ((T - 8) // 8), T):
        P, Psnap, S = light_step(t, P, s, Psnap, S)
    out_ref[...] = S + jnp.log(_dot(Psnap * expend, BIT))


def _tc_denominator(em_t, lens2, tlogt, start_col, end_col):
    return pl.pallas_call(
        _den_body,
        out_shape=jax.ShapeDtypeStruct((B, C), _F32),
        in_specs=[pl.BlockSpec((T, B, CK), lambda: (0, 0, 0)),
                  pl.BlockSpec((B, 1), lambda: (0, 0)),
                  pl.BlockSpec((CK, CK), lambda: (0, 0)),
                  pl.BlockSpec((1, CK), lambda: (0, 0)),
                  pl.BlockSpec((1, CK), lambda: (0, 0))],
        out_specs=pl.BlockSpec((B, C), lambda: (0, 0)),
    )(em_t, lens2, tlogt, start_col, end_col)


@jax.jit
def kernel(emissions, tags, token_sizes, transitions, start_transitions,
           end_transitions):
    em = emissions.reshape(T, B, CK)
    tg32 = tags.astype(_I32)
    lens = token_sizes.astype(_I32)
    tlog = jnp.full((CK, CK), -1e30, dtype=_F32)
    for c in range(C):
        sl = slice(c * K, (c + 1) * K)
        tlog = tlog.at[sl, sl].set(transitions[c])
    start_row = start_transitions.reshape(1, CK)
    end_row = end_transitions.reshape(1, CK)

    lens_x = jnp.repeat(lens, C)
    partials = _sc_numerator(emissions.reshape(-1), tg32.reshape(-1), lens_x,
                             transitions.reshape(-1).astype(_F32),
                             start_transitions.reshape(-1),
                             end_transitions.reshape(-1))
    den = _tc_denominator(em, lens.reshape(B, 1), tlog, start_row, end_row)
    num = partials.sum(axis=0).reshape(B, C)
    return num - den


# speedup vs baseline: 1.1225x; 1.0433x over previous
import functools

import jax
import jax.numpy as jnp
from jax import lax
from jax.experimental import pallas as pl
from jax.experimental.pallas import tpu as pltpu
from jax.experimental.pallas import tpu_sc as plsc

T, B, C, K = 512, 16, 4, 64
CK = C * K
NW = 32
TPW = T // NW
LG = 4

_F32 = jnp.float32
_I32 = jnp.int32


def _dot(a, b):
    return jnp.dot(a, b, preferred_element_type=_F32)


def _dotbf(a, b):
    return jnp.dot(a.astype(jnp.bfloat16), b.astype(jnp.bfloat16),
                   preferred_element_type=_F32)


def _num_body(em_hbm, tags_hbm, lens_hbm, trans_hbm, start_hbm, end_hbm,
              out_hbm, em_v, tg_v, trans_v, start_v, end_v, lens_v, acc_v):
    cid = lax.axis_index("c")
    sid = lax.axis_index("s")
    w = sid * 2 + cid
    t0 = w * TPW
    pltpu.sync_copy(em_hbm.at[pl.ds(w * (TPW * B * C * K), TPW * B * C * K)],
                    em_v)
    prev_off = pl.multiple_of(jnp.maximum(w * (B * C) * TPW - B * C, 0), B * C)
    pltpu.sync_copy(tags_hbm.at[pl.ds(prev_off, B * C)], tg_v.at[pl.ds(0, B * C)])
    pltpu.sync_copy(tags_hbm.at[pl.ds(w * (B * C) * TPW, B * C * TPW)],
                    tg_v.at[pl.ds(B * C, B * C * TPW)])
    pltpu.sync_copy(trans_hbm, trans_v)
    pltpu.sync_copy(start_hbm, start_v)
    pltpu.sync_copy(end_hbm, end_v)
    pltpu.sync_copy(lens_hbm, lens_v)

    lane = lax.broadcasted_iota(_I32, (16,), 0)
    consts = []
    for g in range(LG):
        p = lane + g * 16
        c = p & 3
        lens_g = lens_v[pl.ds(g * 16, 16)]
        consts.append((p * K, c * (K * K), c * K, lens_g))

    def step(t_l, accs):
        t_g = t0 + t_l
        out = []
        for g in range(LG):
            p64, c4096, c64, lens_g = consts[g]
            base = t_l * (B * C) + g * 16
            tgc = tg_v[pl.ds(base + B * C, 16)]
            tgp = tg_v[pl.ds(base, 16)]
            e = plsc.load_gather(em_v, [t_l * (B * C * K) + p64 + tgc])
            tr = plsc.load_gather(trans_v, [c4096 + tgp * K + tgc])
            en = plsc.load_gather(end_v, [c64 + tgc])
            st = plsc.load_gather(start_v, [c64 + tgc])
            maskf = jnp.where(t_g < lens_g, 1.0, 0.0).astype(_F32)
            endf = jnp.where(t_g == lens_g - 1, 1.0, 0.0).astype(_F32)
            live = jnp.where(t_g == 0, e + st, (e + tr) * maskf)
            out.append(accs[g] + live + en * endf)
        return tuple(out)

    zero = jnp.zeros((16,), _F32)
    accs = lax.fori_loop(0, TPW, step, (zero,) * LG)
    for g in range(LG):
        acc_v[pl.ds(g * 16, 16)] = accs[g]
    pltpu.sync_copy(acc_v, out_hbm.at[w])


def _sc_numerator(em_flat, tags_flat, lens, trans_flat, start_flat, end_flat):
    mesh = plsc.VectorSubcoreMesh(core_axis_name="c", subcore_axis_name="s")
    kern = functools.partial(
        pl.kernel,
        mesh=mesh,
        compiler_params=pltpu.CompilerParams(needs_layout_passes=False),
        out_type=jax.ShapeDtypeStruct((NW, B * C), _F32),
        scratch_types=[
            pltpu.VMEM((TPW * B * C * K,), _F32),
            pltpu.VMEM((B * C * (TPW + 1),), _I32),
            pltpu.VMEM((C * K * K,), _F32),
            pltpu.VMEM((C * K,), _F32),
            pltpu.VMEM((C * K,), _F32),
            pltpu.VMEM((B * C,), _I32),
            pltpu.VMEM((B * C,), _F32),
        ],
    )(_num_body)
    return kern(em_flat, tags_flat, lens, trans_flat, start_flat, end_flat)


def _den_body(em_ref, lens_ref, tlog_ref, start_ref, end_ref, out_ref):
    lane_c = lax.broadcasted_iota(_I32, (C, CK), 1) // K
    row_c = lax.broadcasted_iota(_I32, (C, CK), 0)
    BI = (lane_c == row_c).astype(_F32)
    BIT = BI.T

    E = jnp.exp(tlog_ref[...]).astype(jnp.bfloat16)
    expend = jnp.exp(end_ref[...])
    lens = lens_ref[...]

    P = jnp.exp(start_ref[...] + em_ref[0])
    norm = _dotbf(P, BIT)
    P = P * (1.0 / _dotbf(norm, BI))
    s = jnp.log(norm)
    Psnap = P
    S = s

    def light_step(t, P, s, Psnap, S):
        Pn = _dotbf(P, E) * jnp.exp(em_ref[t])
        hit = t == lens - 1
        Psnap = jnp.where(hit, Pn, Psnap)
        S = jnp.where(hit, s, S)
        return Pn, Psnap, S

    def block(i, carry):
        P, s, Psnap, S = carry
        norm = _dotbf(P, BIT)
        invb = 1.0 / _dotbf(norm, BI)
        t0 = 1 + i * 8
        for j in range(8):
            P, Psnap, S = light_step(t0 + j, P, s, Psnap, S)
        P = P * invb
        s = s + jnp.log(norm)
        return P, s, Psnap, S

    P, s, Psnap, S = lax.fori_loop(0, (T - 8) // 8, block, (P, s, Psnap, S),
                                   unroll=3)
    for t in range(1 + 8 * ((T - 8) // 8), T):
        P, Psnap, S = light_step(t, P, s, Psnap, S)
    out_ref[...] = S + jnp.log(_dot(Psnap * expend, BIT))


def _tc_denominator(em_t, lens2, tlogt, start_col, end_col):
    return pl.pallas_call(
        _den_body,
        out_shape=jax.ShapeDtypeStruct((B, C), _F32),
        in_specs=[pl.BlockSpec((T, B, CK), lambda: (0, 0, 0)),
                  pl.BlockSpec((B, 1), lambda: (0, 0)),
                  pl.BlockSpec((CK, CK), lambda: (0, 0)),
                  pl.BlockSpec((1, CK), lambda: (0, 0)),
                  pl.BlockSpec((1, CK), lambda: (0, 0))],
        out_specs=pl.BlockSpec((B, C), lambda: (0, 0)),
    )(em_t, lens2, tlogt, start_col, end_col)


@jax.jit
def kernel(emissions, tags, token_sizes, transitions, start_transitions,
           end_transitions):
    em = emissions.reshape(T, B, CK)
    tg32 = tags.astype(_I32)
    lens = token_sizes.astype(_I32)
    tlog = jnp.full((CK, CK), -1e30, dtype=_F32)
    for c in range(C):
        sl = slice(c * K, (c + 1) * K)
        tlog = tlog.at[sl, sl].set(transitions[c])
    start_row = start_transitions.reshape(1, CK)
    end_row = end_transitions.reshape(1, CK)

    lens_x = jnp.repeat(lens, C)
    partials = _sc_numerator(emissions.reshape(-1), tg32.reshape(-1), lens_x,
                             transitions.reshape(-1).astype(_F32),
                             start_transitions.reshape(-1),
                             end_transitions.reshape(-1))
    den = _tc_denominator(em, lens.reshape(B, 1), tlog, start_row, end_row)
    num = partials.sum(axis=0).reshape(B, C)
    return num - den
